# Initial kernel scaffold; baseline (speedup 1.0000x reference)
import jax
import jax.numpy as jnp
from jax import lax
from jax.experimental import pallas as pl
from jax.experimental.pallas import tpu as pltpu
from jax.experimental.pallas import tpu_sc as plsc

_N = 50000
_E = 1600000
_G = 64
_IN_F = 3
_HID = 16
_OUT_F = 32
_NCLS = 10
_K = 4
_EPS = 1e-5

_NC = 2
_NS = 16
_NW = _NC * _NS
_EPW = _E // _NW
_C = 80
_NCH = _EPW // _C

_RPT = 3200
_RTL = _N - _RPT * (_NS - 1)

_NB = 2000
_NBLK = _N // _NB
_EBB = 3200
_EBLK = _E // _EBB


def _basis_body(ea_ref, b_ref):
    f = jnp.clip(ea_ref[...], 0.0, 1.0)
    f0 = f[:, 0:1]
    f1 = f[:, 1:2]
    g0 = 1.0 - f0
    g1 = 1.0 - f1
    b_ref[...] = jnp.concatenate([g0 * g1, f0 * g1, g0 * f1, f0 * f1], axis=1)


_basis_call = pl.pallas_call(
    _basis_body,
    grid=(_EBLK,),
    in_specs=[pl.BlockSpec((_EBB, 2), lambda i: (i, 0))],
    out_specs=pl.BlockSpec((_EBB, 4), lambda i: (i, 0)),
    out_shape=jax.ShapeDtypeStruct((_E, 4), jnp.float32),
)


def _xw_body(x_ref, w_ref, y_ref):
    y_ref[...] = jnp.dot(x_ref[...], w_ref[...],
                         preferred_element_type=jnp.float32)


_xw_call = pl.pallas_call(
    _xw_body,
    grid=(_NBLK,),
    in_specs=[pl.BlockSpec((_NB, _IN_F), lambda i: (i, 0)),
              pl.BlockSpec((_IN_F, _K * _HID), lambda i: (0, 0))],
    out_specs=pl.BlockSpec((_NB, _K * _HID), lambda i: (i, 0)),
    out_shape=jax.ShapeDtypeStruct((_N, _K * _HID), jnp.float32),
)


def _make_combine_stats(F):
    def body(acc_ref, deg_ref, h_ref, st_ref):
        i = pl.program_id(0)
        acc = acc_ref[0] + acc_ref[1]
        deg = deg_ref[0, 0, 0, :] + deg_ref[1, 0, 0, :]
        deg = jnp.maximum(deg, 1.0)
        h = acc / deg[:, None]
        h = jnp.where(h > 0.0, h, jnp.exp(h) - 1.0)
        h_ref[...] = h

        @pl.when(i == 0)
        def _():
            st_ref[...] = jnp.zeros((2, F), jnp.float32)

        st_ref[0:1, :] += jnp.sum(h, axis=0, keepdims=True)
        st_ref[1:2, :] += jnp.sum(h * h, axis=0, keepdims=True)

    return pl.pallas_call(
        body,
        grid=(_NBLK,),
        in_specs=[pl.BlockSpec((2, _NB, F), lambda i: (0, i, 0)),
                  pl.BlockSpec((2, 1, 1, _NB), lambda i: (0, i, 0, 0))],
        out_specs=[pl.BlockSpec((_NB, F), lambda i: (i, 0)),
                   pl.BlockSpec((2, F), lambda i: (0, 0))],
        out_shape=[jax.ShapeDtypeStruct((_N, F), jnp.float32),
                   jax.ShapeDtypeStruct((2, F), jnp.float32)],
    )


_cs16 = _make_combine_stats(_HID)
_cs32 = _make_combine_stats(_OUT_F)


def _m2_body(h_ref, st_ref, g_ref, b_ref, w_ref, y_ref):
    mean = st_ref[0:1, :] * (1.0 / _N)
    var = st_ref[1:2, :] * (1.0 / _N) - mean * mean
    scale = g_ref[...] * lax.rsqrt(var + _EPS)
    shift = b_ref[...] - mean * scale
    hb = h_ref[...] * scale + shift
    y_ref[...] = jnp.dot(hb, w_ref[...], preferred_element_type=jnp.float32)


_m2_call = pl.pallas_call(
    _m2_body,
    grid=(_NBLK,),
    in_specs=[pl.BlockSpec((_NB, _HID), lambda i: (i, 0)),
              pl.BlockSpec((2, _HID), lambda i: (0, 0)),
              pl.BlockSpec((1, _HID), lambda i: (0, 0)),
              pl.BlockSpec((1, _HID), lambda i: (0, 0)),
              pl.BlockSpec((_HID, _K * _OUT_F), lambda i: (0, 0))],
    out_specs=pl.BlockSpec((_NB, _K * _OUT_F), lambda i: (i, 0)),
    out_shape=jax.ShapeDtypeStruct((_N, _K * _OUT_F), jnp.float32),
)


def _f2_body(h_ref, st_ref, g_ref, b_ref, batch_ref, wfc_ref, o_ref,
             ps_ref, cnt_ref):
    i = pl.program_id(0)
    mean = st_ref[0:1, :] * (1.0 / _N)
    var = st_ref[1:2, :] * (1.0 / _N) - mean * mean
    scale = g_ref[...] * lax.rsqrt(var + _EPS)
    shift = b_ref[...] - mean * scale
    hb = h_ref[...] * scale + shift
    bt = batch_ref[0, 0, :]
    oh = (bt[:, None] ==
          lax.broadcasted_iota(jnp.int32, (_NB, _G), 1)).astype(jnp.float32)

    @pl.when(i == 0)
    def _():
        ps_ref[...] = jnp.zeros((_G, _OUT_F), jnp.float32)
        cnt_ref[...] = jnp.zeros((_G, _OUT_F), jnp.float32)

    dn = (((0,), (0,)), ((), ()))
    ps_ref[...] += lax.dot_general(oh, hb, dn,
                                   preferred_element_type=jnp.float32)
    cnt_ref[...] += lax.dot_general(oh, jnp.ones((_NB, _OUT_F), jnp.float32),
                                    dn, preferred_element_type=jnp.float32)

    @pl.when(i == _NBLK - 1)
    def _():
        pooled = ps_ref[...] / jnp.maximum(cnt_ref[...], 1.0)
        o_ref[...] = jnp.dot(pooled, wfc_ref[...],
                             preferred_element_type=jnp.float32)


_f2_call = pl.pallas_call(
    _f2_body,
    grid=(_NBLK,),
    in_specs=[pl.BlockSpec((_NB, _OUT_F), lambda i: (i, 0)),
              pl.BlockSpec((2, _OUT_F), lambda i: (0, 0)),
              pl.BlockSpec((1, _OUT_F), lambda i: (0, 0)),
              pl.BlockSpec((1, _OUT_F), lambda i: (0, 0)),
              pl.BlockSpec((1, 1, _NB), lambda i: (i, 0, 0)),
              pl.BlockSpec((_OUT_F, _NCLS), lambda i: (0, 0))],
    out_specs=pl.BlockSpec((_G, _NCLS), lambda i: (0, 0)),
    out_shape=jax.ShapeDtypeStruct((_G, _NCLS), jnp.float32),
    scratch_shapes=[pltpu.VMEM((_G, _OUT_F), jnp.float32),
                    pltpu.VMEM((_G, _OUT_F), jnp.float32)],
)


def _make_edge_call(F, with_deg):
    KF = _K * F
    mesh = plsc.VectorSubcoreMesh(core_axis_name="c", subcore_axis_name="s")
    if with_deg:
        out_type = (jax.ShapeDtypeStruct((_NC, _N, F), jnp.float32),
                    jax.ShapeDtypeStruct((_NC, _N), jnp.float32))
    else:
        out_type = jax.ShapeDtypeStruct((_NC, _N, F), jnp.float32)

    scratch = [
        pltpu.VMEM((2, _C), jnp.int32),
        pltpu.VMEM((_C, _K), jnp.float32),
        pltpu.VMEM((_C, KF), jnp.float32),
        pltpu.VMEM((_C, F), jnp.float32),
        pltpu.VMEM((_C,), jnp.float32),
        pltpu.VMEM_SHARED((_N, F), jnp.float32),
    ]
    if with_deg:
        scratch.append(pltpu.VMEM_SHARED((_N,), jnp.float32))
    scratch.append(pltpu.SemaphoreType.DMA)

    def body(*refs):
        if with_deg:
            (ei, basis, y, zacc, zdeg, acc_out, deg_out,
             ei_v, b_v, rows_v, msg_v, ones_v, acc_sh, deg_sh, sem) = refs
        else:
            (ei, basis, y, zacc, acc_out,
             ei_v, b_v, rows_v, msg_v, ones_v, acc_sh, sem) = refs
        cid = lax.axis_index("c")
        sid = lax.axis_index("s")
        wid = sid * _NC + cid

        for k in range(_C // 16):
            ones_v[pl.ds(k * 16, 16)] = jnp.ones((16,), jnp.float32)

        @pl.when(sid < _NS - 1)
        def _():
            s0 = sid * _RPT
            pltpu.sync_copy(zacc.at[pl.ds(s0, _RPT)],
                            acc_sh.at[pl.ds(s0, _RPT)])
            if with_deg:
                pltpu.sync_copy(zdeg.at[pl.ds(s0, _RPT)],
                                deg_sh.at[pl.ds(s0, _RPT)])

        @pl.when(sid == _NS - 1)
        def _():
            s0 = (_NS - 1) * _RPT
            pltpu.sync_copy(zacc.at[pl.ds(s0, _RTL)],
                            acc_sh.at[pl.ds(s0, _RTL)])
            if with_deg:
                pltpu.sync_copy(zdeg.at[pl.ds(s0, _RTL)],
                                deg_sh.at[pl.ds(s0, _RTL)])

        plsc.subcore_barrier()

        ebase = wid * _EPW

        def chunk(i, carry):
            eb = ebase + i * _C
            pltpu.sync_copy(ei.at[:, pl.ds(eb, _C)], ei_v)
            pltpu.sync_copy(basis.at[pl.ds(eb, _C)], b_v)
            pltpu.async_copy(y.at[ei_v.at[0]], rows_v, sem).wait()

            def edge(e, c2):
                b0 = b_v[e, 0]
                b1 = b_v[e, 1]
                b2 = b_v[e, 2]
                b3 = b_v[e, 3]
                for h in range(F // 16):
                    m = rows_v[e, pl.ds(h * 16, 16)] * b0
                    m = m + rows_v[e, pl.ds(F + h * 16, 16)] * b1
                    m = m + rows_v[e, pl.ds(2 * F + h * 16, 16)] * b2
                    m = m + rows_v[e, pl.ds(3 * F + h * 16, 16)] * b3
                    msg_v[e, pl.ds(h * 16, 16)] = m
                return c2

            lax.fori_loop(0, _C, edge, 0)
            pltpu.sync_copy(msg_v, acc_sh.at[ei_v.at[1]], add=True)
            if with_deg:
                pltpu.sync_copy(ones_v, deg_sh.at[ei_v.at[1]], add=True)
            return carry

        lax.fori_loop(0, _NCH, chunk, 0)
        plsc.subcore_barrier()

        @pl.when(sid < _NS - 1)
        def _():
            s0 = sid * _RPT
            pltpu.sync_copy(acc_sh.at[pl.ds(s0, _RPT)],
                            acc_out.at[cid, pl.ds(s0, _RPT)])
            if with_deg:
                pltpu.sync_copy(deg_sh.at[pl.ds(s0, _RPT)],
                                deg_out.at[cid, pl.ds(s0, _RPT)])

        @pl.when(sid == _NS - 1)
        def _():
            s0 = (_NS - 1) * _RPT
            pltpu.sync_copy(acc_sh.at[pl.ds(s0, _RTL)],
                            acc_out.at[cid, pl.ds(s0, _RTL)])
            if with_deg:
                pltpu.sync_copy(deg_sh.at[pl.ds(s0, _RTL)],
                                deg_out.at[cid, pl.ds(s0, _RTL)])

    return pl.kernel(body, out_type=out_type, mesh=mesh,
                     scratch_types=scratch)


_edge1 = _make_edge_call(_HID, True)
_edge2 = _make_edge_call(_OUT_F, False)


def kernel(x, edge_index, edge_attr, batch, W1, gamma1, beta1,
           W2, gamma2, beta2, Wfc):
    f32 = jnp.float32
    basis = _basis_call(edge_attr)
    w1f = W1.transpose(1, 0, 2).reshape(_IN_F, _K * _HID)
    w2f = W2.transpose(1, 0, 2).reshape(_HID, _K * _OUT_F)
    y1 = _xw_call(x, w1f)

    z1 = jnp.zeros((_N, _HID), f32)
    zd = jnp.zeros((_N,), f32)
    acc1p, degp = _edge1(edge_index, basis, y1, z1, zd)
    degr = degp.reshape(_NC, _NBLK, 1, _NB)

    h1, st1 = _cs16(acc1p, degr)
    y2 = _m2_call(h1, st1, gamma1.reshape(1, _HID), beta1.reshape(1, _HID),
                  w2f)

    z2 = jnp.zeros((_N, _OUT_F), f32)
    acc2p = _edge2(edge_index, basis, y2, z2)

    h2, st2 = _cs32(acc2p, degr)
    out = _f2_call(h2, st2, gamma2.reshape(1, _OUT_F),
                   beta2.reshape(1, _OUT_F),
                   batch.reshape(_NBLK, 1, _NB), Wfc)
    return out


# speedup vs baseline: 9.6521x; 9.6521x over previous
import jax
import jax.numpy as jnp
from jax import lax
from jax.experimental import pallas as pl
from jax.experimental.pallas import tpu as pltpu
from jax.experimental.pallas import tpu_sc as plsc

_N = 50000
_E = 1600000
_G = 64
_IN_F = 3
_HID = 16
_OUT_F = 32
_NCLS = 10
_K = 4
_EPS = 1e-5

_NC = 2
_NS = 16
_NW = _NC * _NS
_EPW = _E // _NW
_C = 80
_NCH = _EPW // _C

_RPT = 3200
_RTL = _N - _RPT * (_NS - 1)

_NB = 2000
_NBLK = _N // _NB
_EBB = 3200
_EBLK = _E // _EBB


def _basis_body(ea_ref, b_ref):
    f = jnp.clip(ea_ref[...], 0.0, 1.0)
    f0 = f[0:1, :]
    f1 = f[1:2, :]
    g0 = 1.0 - f0
    g1 = 1.0 - f1
    b_ref[...] = jnp.concatenate([g0 * g1, f0 * g1, g0 * f1, f0 * f1], axis=0)


_basis_call = pl.pallas_call(
    _basis_body,
    grid=(_EBLK,),
    in_specs=[pl.BlockSpec((2, _EBB), lambda i: (0, i))],
    out_specs=pl.BlockSpec((4, _EBB), lambda i: (0, i)),
    out_shape=jax.ShapeDtypeStruct((4, _E), jnp.float32),
)


def _xw_body(x_ref, w_ref, y_ref):
    y_ref[...] = jnp.dot(x_ref[...], w_ref[...],
                         preferred_element_type=jnp.float32)


_xw_call = pl.pallas_call(
    _xw_body,
    grid=(_NBLK,),
    in_specs=[pl.BlockSpec((_NB, _IN_F), lambda i: (i, 0)),
              pl.BlockSpec((_IN_F, _K * _HID), lambda i: (0, 0))],
    out_specs=pl.BlockSpec((_NB, _K * _HID), lambda i: (i, 0)),
    out_shape=jax.ShapeDtypeStruct((_N, _K * _HID), jnp.float32),
)


def _make_combine_stats(F):
    def body(acc_ref, deg_ref, h_ref, st_ref):
        i = pl.program_id(0)
        acc = acc_ref[0] + acc_ref[1]
        deg = deg_ref[0, 0, 0, :] + deg_ref[1, 0, 0, :]
        deg = jnp.maximum(deg, 1.0)
        h = acc / deg[:, None]
        h = jnp.where(h > 0.0, h, jnp.exp(h) - 1.0)
        h_ref[...] = h

        @pl.when(i == 0)
        def _():
            st_ref[...] = jnp.zeros((2, F), jnp.float32)

        st_ref[0:1, :] += jnp.sum(h, axis=0, keepdims=True)
        st_ref[1:2, :] += jnp.sum(h * h, axis=0, keepdims=True)

    return pl.pallas_call(
        body,
        grid=(_NBLK,),
        in_specs=[pl.BlockSpec((2, _NB, F), lambda i: (0, i, 0)),
                  pl.BlockSpec((2, 1, 1, _NB), lambda i: (0, i, 0, 0))],
        out_specs=[pl.BlockSpec((_NB, F), lambda i: (i, 0)),
                   pl.BlockSpec((2, F), lambda i: (0, 0))],
        out_shape=[jax.ShapeDtypeStruct((_N, F), jnp.float32),
                   jax.ShapeDtypeStruct((2, F), jnp.float32)],
    )


_cs16 = _make_combine_stats(_HID)
_cs32 = _make_combine_stats(_OUT_F)


def _m2_body(h_ref, st_ref, g_ref, b_ref, w_ref, y_ref):
    mean = st_ref[0:1, :] * (1.0 / _N)
    var = st_ref[1:2, :] * (1.0 / _N) - mean * mean
    scale = g_ref[...] * lax.rsqrt(var + _EPS)
    shift = b_ref[...] - mean * scale
    hb = h_ref[...] * scale + shift
    y_ref[...] = jnp.dot(hb, w_ref[...], preferred_element_type=jnp.float32)


_m2_call = pl.pallas_call(
    _m2_body,
    grid=(_NBLK,),
    in_specs=[pl.BlockSpec((_NB, _HID), lambda i: (i, 0)),
              pl.BlockSpec((2, _HID), lambda i: (0, 0)),
              pl.BlockSpec((1, _HID), lambda i: (0, 0)),
              pl.BlockSpec((1, _HID), lambda i: (0, 0)),
              pl.BlockSpec((_HID, _K * _OUT_F), lambda i: (0, 0))],
    out_specs=pl.BlockSpec((_NB, _K * _OUT_F), lambda i: (i, 0)),
    out_shape=jax.ShapeDtypeStruct((_N, _K * _OUT_F), jnp.float32),
)


def _f2_body(h_ref, st_ref, g_ref, b_ref, batch_ref, wfc_ref, o_ref,
             ps_ref, cnt_ref):
    i = pl.program_id(0)
    mean = st_ref[0:1, :] * (1.0 / _N)
    var = st_ref[1:2, :] * (1.0 / _N) - mean * mean
    scale = g_ref[...] * lax.rsqrt(var + _EPS)
    shift = b_ref[...] - mean * scale
    hb = h_ref[...] * scale + shift
    bt = batch_ref[0, 0, :]
    oh = (bt[:, None] ==
          lax.broadcasted_iota(jnp.int32, (_NB, _G), 1)).astype(jnp.float32)

    @pl.when(i == 0)
    def _():
        ps_ref[...] = jnp.zeros((_G, _OUT_F), jnp.float32)
        cnt_ref[...] = jnp.zeros((_G, _OUT_F), jnp.float32)

    dn = (((0,), (0,)), ((), ()))
    ps_ref[...] += lax.dot_general(oh, hb, dn,
                                   preferred_element_type=jnp.float32)
    cnt_ref[...] += lax.dot_general(oh, jnp.ones((_NB, _OUT_F), jnp.float32),
                                    dn, preferred_element_type=jnp.float32)

    @pl.when(i == _NBLK - 1)
    def _():
        pooled = ps_ref[...] / jnp.maximum(cnt_ref[...], 1.0)
        o_ref[...] = jnp.dot(pooled, wfc_ref[...],
                             preferred_element_type=jnp.float32)


_f2_call = pl.pallas_call(
    _f2_body,
    grid=(_NBLK,),
    in_specs=[pl.BlockSpec((_NB, _OUT_F), lambda i: (i, 0)),
              pl.BlockSpec((2, _OUT_F), lambda i: (0, 0)),
              pl.BlockSpec((1, _OUT_F), lambda i: (0, 0)),
              pl.BlockSpec((1, _OUT_F), lambda i: (0, 0)),
              pl.BlockSpec((1, 1, _NB), lambda i: (i, 0, 0)),
              pl.BlockSpec((_OUT_F, _NCLS), lambda i: (0, 0))],
    out_specs=pl.BlockSpec((_G, _NCLS), lambda i: (0, 0)),
    out_shape=jax.ShapeDtypeStruct((_G, _NCLS), jnp.float32),
    scratch_shapes=[pltpu.VMEM((_G, _OUT_F), jnp.float32),
                    pltpu.VMEM((_G, _OUT_F), jnp.float32)],
)


def _make_edge_call(F, with_deg):
    KF = _K * F
    mesh = plsc.VectorSubcoreMesh(core_axis_name="c", subcore_axis_name="s",
                                  num_cores=_NC, num_subcores=_NS)
    if with_deg:
        out_type = (jax.ShapeDtypeStruct((_NC, _N, F), jnp.float32),
                    jax.ShapeDtypeStruct((_NC, _N), jnp.float32))
    else:
        out_type = jax.ShapeDtypeStruct((_NC, _N, F), jnp.float32)

    scratch = [
        pltpu.VMEM((2, _C), jnp.int32),
        pltpu.VMEM((_K, _C), jnp.float32),
        pltpu.VMEM((_C, KF), jnp.float32),
        pltpu.VMEM((_C, F), jnp.float32),
        pltpu.VMEM((_C,), jnp.float32),
        pltpu.VMEM_SHARED((_N, F), jnp.float32),
    ]
    if with_deg:
        scratch.append(pltpu.VMEM_SHARED((_N,), jnp.float32))
    scratch.append(pltpu.SemaphoreType.DMA)

    def body(*refs):
        if with_deg:
            (ei, basis, y, zacc, zdeg, acc_out, deg_out,
             ei_v, b_v, rows_v, msg_v, ones_v, acc_sh, deg_sh, sem) = refs
        else:
            (ei, basis, y, zacc, acc_out,
             ei_v, b_v, rows_v, msg_v, ones_v, acc_sh, sem) = refs
        cid = lax.axis_index("c")
        sid = lax.axis_index("s")
        wid = sid * _NC + cid

        for k in range(_C // 16):
            ones_v[pl.ds(k * 16, 16)] = jnp.ones((16,), jnp.float32)

        @pl.when(sid < _NS - 1)
        def _():
            s0 = sid * _RPT
            pltpu.sync_copy(zacc.at[pl.ds(s0, _RPT)],
                            acc_sh.at[pl.ds(s0, _RPT)])
            if with_deg:
                pltpu.sync_copy(zdeg.at[pl.ds(s0, _RPT)],
                                deg_sh.at[pl.ds(s0, _RPT)])

        @pl.when(sid == _NS - 1)
        def _():
            s0 = (_NS - 1) * _RPT
            pltpu.sync_copy(zacc.at[pl.ds(s0, _RTL)],
                            acc_sh.at[pl.ds(s0, _RTL)])
            if with_deg:
                pltpu.sync_copy(zdeg.at[pl.ds(s0, _RTL)],
                                deg_sh.at[pl.ds(s0, _RTL)])

        plsc.subcore_barrier()

        ebase = wid * _EPW

        def chunk(i, carry):
            eb = ebase + i * _C
            pltpu.sync_copy(ei.at[:, pl.ds(eb, _C)], ei_v)
            pltpu.sync_copy(basis.at[:, pl.ds(eb, _C)], b_v)
            pltpu.async_copy(y.at[ei_v.at[0]], rows_v, sem).wait()

            def group(g, c2):
                g16 = g * 16
                bb0 = b_v[0, pl.ds(g16, 16)]
                bb1 = b_v[1, pl.ds(g16, 16)]
                bb2 = b_v[2, pl.ds(g16, 16)]
                bb3 = b_v[3, pl.ds(g16, 16)]
                for j in range(16):
                    e = g16 + j
                    for h in range(F // 16):
                        m = rows_v[e, pl.ds(h * 16, 16)] * bb0[j]
                        m = m + rows_v[e, pl.ds(F + h * 16, 16)] * bb1[j]
                        m = m + rows_v[e, pl.ds(2 * F + h * 16, 16)] * bb2[j]
                        m = m + rows_v[e, pl.ds(3 * F + h * 16, 16)] * bb3[j]
                        msg_v[e, pl.ds(h * 16, 16)] = m
                return c2

            lax.fori_loop(0, _C // 16, group, 0)
            pltpu.sync_copy(msg_v, acc_sh.at[ei_v.at[1]], add=True)
            if with_deg:
                pltpu.sync_copy(ones_v, deg_sh.at[ei_v.at[1]], add=True)
            return carry

        lax.fori_loop(0, _NCH, chunk, 0)
        plsc.subcore_barrier()

        @pl.when(sid < _NS - 1)
        def _():
            s0 = sid * _RPT
            pltpu.sync_copy(acc_sh.at[pl.ds(s0, _RPT)],
                            acc_out.at[cid, pl.ds(s0, _RPT)])
            if with_deg:
                pltpu.sync_copy(deg_sh.at[pl.ds(s0, _RPT)],
                                deg_out.at[cid, pl.ds(s0, _RPT)])

        @pl.when(sid == _NS - 1)
        def _():
            s0 = (_NS - 1) * _RPT
            pltpu.sync_copy(acc_sh.at[pl.ds(s0, _RTL)],
                            acc_out.at[cid, pl.ds(s0, _RTL)])
            if with_deg:
                pltpu.sync_copy(deg_sh.at[pl.ds(s0, _RTL)],
                                deg_out.at[cid, pl.ds(s0, _RTL)])

    return pl.kernel(body, out_type=out_type, mesh=mesh,
                     scratch_types=scratch,
                     compiler_params=pltpu.CompilerParams(
                         use_tc_tiling_on_sc=False))


_edge1 = _make_edge_call(_HID, True)
_edge2 = _make_edge_call(_OUT_F, False)


def kernel(x, edge_index, edge_attr, batch, W1, gamma1, beta1,
           W2, gamma2, beta2, Wfc):
    f32 = jnp.float32
    basis = _basis_call(edge_attr.T)
    w1f = W1.transpose(1, 0, 2).reshape(_IN_F, _K * _HID)
    w2f = W2.transpose(1, 0, 2).reshape(_HID, _K * _OUT_F)
    y1 = _xw_call(x, w1f)

    z1 = jnp.zeros((_N, _HID), f32)
    zd = jnp.zeros((_N,), f32)
    acc1p, degp = _edge1(edge_index, basis, y1, z1, zd)
    degr = degp.reshape(_NC, _NBLK, 1, _NB)

    h1, st1 = _cs16(acc1p, degr)
    y2 = _m2_call(h1, st1, gamma1.reshape(1, _HID), beta1.reshape(1, _HID),
                  w2f)

    z2 = jnp.zeros((_N, _OUT_F), f32)
    acc2p = _edge2(edge_index, basis, y2, z2)

    h2, st2 = _cs32(acc2p, degr)
    out = _f2_call(h2, st2, gamma2.reshape(1, _OUT_F),
                   beta2.reshape(1, _OUT_F),
                   batch.reshape(_NBLK, 1, _NB), Wfc)
    return out


# speedup vs baseline: 15.5816x; 1.6143x over previous
import jax
import jax.numpy as jnp
from jax import lax
from jax.experimental import pallas as pl
from jax.experimental.pallas import tpu as pltpu
from jax.experimental.pallas import tpu_sc as plsc

_N = 50000
_E = 1600000
_G = 64
_IN_F = 3
_HID = 16
_OUT_F = 32
_NCLS = 10
_K = 4
_EPS = 1e-5

_NC = 2
_NS = 16
_NW = _NC * _NS
_EPW = _E // _NW
_C = 80
_NCH = _EPW // _C

_RPT = 3200
_RTL = _N - _RPT * (_NS - 1)

_NB = 2000
_NBLK = _N // _NB
_EBB = 3200
_EBLK = _E // _EBB


def _basis_body(ea_ref, b_ref):
    f = jnp.clip(ea_ref[...], 0.0, 1.0)
    f0 = f[0:1, :]
    f1 = f[1:2, :]
    g0 = 1.0 - f0
    g1 = 1.0 - f1
    b_ref[...] = jnp.concatenate([g0 * g1, f0 * g1, g0 * f1, f0 * f1], axis=0)


_basis_call = pl.pallas_call(
    _basis_body,
    grid=(_EBLK,),
    in_specs=[pl.BlockSpec((2, _EBB), lambda i: (0, i))],
    out_specs=pl.BlockSpec((4, _EBB), lambda i: (0, i)),
    out_shape=jax.ShapeDtypeStruct((4, _E), jnp.float32),
)


def _xw_body(x_ref, w_ref, y_ref):
    y_ref[...] = jnp.dot(x_ref[...], w_ref[...],
                         preferred_element_type=jnp.float32)


_xw_call = pl.pallas_call(
    _xw_body,
    grid=(_NBLK,),
    in_specs=[pl.BlockSpec((_NB, _IN_F), lambda i: (i, 0)),
              pl.BlockSpec((_IN_F, _K * _HID), lambda i: (0, 0))],
    out_specs=pl.BlockSpec((_NB, _K * _HID), lambda i: (i, 0)),
    out_shape=jax.ShapeDtypeStruct((_N, _K * _HID), jnp.float32),
)


def _make_combine_stats(F):
    def body(acc_ref, deg_ref, h_ref, st_ref):
        i = pl.program_id(0)
        acc = acc_ref[0] + acc_ref[1]
        deg = deg_ref[0, 0, 0, :] + deg_ref[1, 0, 0, :]
        deg = jnp.maximum(deg, 1.0)
        h = acc / deg[:, None]
        h = jnp.where(h > 0.0, h, jnp.exp(h) - 1.0)
        h_ref[...] = h

        @pl.when(i == 0)
        def _():
            st_ref[...] = jnp.zeros((2, F), jnp.float32)

        st_ref[0:1, :] += jnp.sum(h, axis=0, keepdims=True)
        st_ref[1:2, :] += jnp.sum(h * h, axis=0, keepdims=True)

    return pl.pallas_call(
        body,
        grid=(_NBLK,),
        in_specs=[pl.BlockSpec((2, _NB, F), lambda i: (0, i, 0)),
                  pl.BlockSpec((2, 1, 1, _NB), lambda i: (0, i, 0, 0))],
        out_specs=[pl.BlockSpec((_NB, F), lambda i: (i, 0)),
                   pl.BlockSpec((2, F), lambda i: (0, 0))],
        out_shape=[jax.ShapeDtypeStruct((_N, F), jnp.float32),
                   jax.ShapeDtypeStruct((2, F), jnp.float32)],
    )


_cs16 = _make_combine_stats(_HID)
_cs32 = _make_combine_stats(_OUT_F)


def _m2_body(h_ref, st_ref, g_ref, b_ref, w_ref, y_ref):
    mean = st_ref[0:1, :] * (1.0 / _N)
    var = st_ref[1:2, :] * (1.0 / _N) - mean * mean
    scale = g_ref[...] * lax.rsqrt(var + _EPS)
    shift = b_ref[...] - mean * scale
    hb = h_ref[...] * scale + shift
    y_ref[...] = jnp.dot(hb, w_ref[...], preferred_element_type=jnp.float32)


_m2_call = pl.pallas_call(
    _m2_body,
    grid=(_NBLK,),
    in_specs=[pl.BlockSpec((_NB, _HID), lambda i: (i, 0)),
              pl.BlockSpec((2, _HID), lambda i: (0, 0)),
              pl.BlockSpec((1, _HID), lambda i: (0, 0)),
              pl.BlockSpec((1, _HID), lambda i: (0, 0)),
              pl.BlockSpec((_HID, _K * _OUT_F), lambda i: (0, 0))],
    out_specs=pl.BlockSpec((_NB, _K * _OUT_F), lambda i: (i, 0)),
    out_shape=jax.ShapeDtypeStruct((_N, _K * _OUT_F), jnp.float32),
)


def _f2_body(h_ref, st_ref, g_ref, b_ref, batch_ref, wfc_ref, o_ref,
             ps_ref, cnt_ref):
    i = pl.program_id(0)
    mean = st_ref[0:1, :] * (1.0 / _N)
    var = st_ref[1:2, :] * (1.0 / _N) - mean * mean
    scale = g_ref[...] * lax.rsqrt(var + _EPS)
    shift = b_ref[...] - mean * scale
    hb = h_ref[...] * scale + shift
    bt = batch_ref[0, 0, :]
    oh = (bt[:, None] ==
          lax.broadcasted_iota(jnp.int32, (_NB, _G), 1)).astype(jnp.float32)

    @pl.when(i == 0)
    def _():
        ps_ref[...] = jnp.zeros((_G, _OUT_F), jnp.float32)
        cnt_ref[...] = jnp.zeros((_G, _OUT_F), jnp.float32)

    dn = (((0,), (0,)), ((), ()))
    ps_ref[...] += lax.dot_general(oh, hb, dn,
                                   preferred_element_type=jnp.float32)
    cnt_ref[...] += lax.dot_general(oh, jnp.ones((_NB, _OUT_F), jnp.float32),
                                    dn, preferred_element_type=jnp.float32)

    @pl.when(i == _NBLK - 1)
    def _():
        pooled = ps_ref[...] / jnp.maximum(cnt_ref[...], 1.0)
        o_ref[...] = jnp.dot(pooled, wfc_ref[...],
                             preferred_element_type=jnp.float32)


_f2_call = pl.pallas_call(
    _f2_body,
    grid=(_NBLK,),
    in_specs=[pl.BlockSpec((_NB, _OUT_F), lambda i: (i, 0)),
              pl.BlockSpec((2, _OUT_F), lambda i: (0, 0)),
              pl.BlockSpec((1, _OUT_F), lambda i: (0, 0)),
              pl.BlockSpec((1, _OUT_F), lambda i: (0, 0)),
              pl.BlockSpec((1, 1, _NB), lambda i: (i, 0, 0)),
              pl.BlockSpec((_OUT_F, _NCLS), lambda i: (0, 0))],
    out_specs=pl.BlockSpec((_G, _NCLS), lambda i: (0, 0)),
    out_shape=jax.ShapeDtypeStruct((_G, _NCLS), jnp.float32),
    scratch_shapes=[pltpu.VMEM((_G, _OUT_F), jnp.float32),
                    pltpu.VMEM((_G, _OUT_F), jnp.float32)],
)


_NBUF = 2
_ER = _E // _C
_RPW = _ER // _NW
_NP = (_RPW - 1) // _NBUF * _NBUF


def _make_edge_call(F, with_deg):
    KF = _K * F
    mesh = plsc.VectorSubcoreMesh(core_axis_name="c", subcore_axis_name="s",
                                  num_cores=_NC, num_subcores=_NS)
    if with_deg:
        out_type = (jax.ShapeDtypeStruct((_NC, _N, F), jnp.float32),
                    jax.ShapeDtypeStruct((_NC, _N), jnp.float32))
    else:
        out_type = jax.ShapeDtypeStruct((_NC, _N, F), jnp.float32)

    scratch = []
    scratch += [pltpu.VMEM((2, _C), jnp.int32)] * _NBUF
    scratch += [pltpu.VMEM((_K, _C), jnp.float32)] * _NBUF
    scratch += [pltpu.VMEM((_C, KF), jnp.float32)] * _NBUF
    scratch += [pltpu.VMEM((_C, F), jnp.float32)] * _NBUF
    scratch += [pltpu.VMEM((_C,), jnp.int32)] * _NBUF
    scratch.append(pltpu.VMEM((_C,), jnp.float32))
    scratch.append(pltpu.VMEM_SHARED((_N, F), jnp.float32))
    if with_deg:
        scratch.append(pltpu.VMEM_SHARED((_N,), jnp.float32))
    nsem = 3 * _NBUF + (_NBUF if with_deg else 0)
    scratch += [pltpu.SemaphoreType.DMA] * nsem

    def body(*refs):
        if with_deg:
            (ei, basis, y, zacc, zdeg, acc_out, deg_out) = refs[:7]
            rest = refs[7:]
        else:
            (ei, basis, y, zacc, acc_out) = refs[:5]
            rest = refs[5:]
        eiv = rest[0:_NBUF]
        bv = rest[_NBUF:2 * _NBUF]
        rows = rest[2 * _NBUF:3 * _NBUF]
        msg = rest[3 * _NBUF:4 * _NBUF]
        dstc = rest[4 * _NBUF:5 * _NBUF]
        ones_v = rest[5 * _NBUF]
        acc_sh = rest[5 * _NBUF + 1]
        pos = 5 * _NBUF + 2
        if with_deg:
            deg_sh = rest[pos]
            pos += 1
        semL = rest[pos:pos + _NBUF]
        semG = rest[pos + _NBUF:pos + 2 * _NBUF]
        semS = rest[pos + 2 * _NBUF:pos + 3 * _NBUF]
        if with_deg:
            semD = rest[pos + 3 * _NBUF:pos + 4 * _NBUF]

        cid = lax.axis_index("c")
        sid = lax.axis_index("s")
        wid = sid * _NC + cid

        for k in range(_C // 16):
            ones_v[pl.ds(k * 16, 16)] = jnp.ones((16,), jnp.float32)

        @pl.when(sid < _NS - 1)
        def _():
            s0 = sid * _RPT
            pltpu.sync_copy(zacc.at[pl.ds(s0, _RPT)],
                            acc_sh.at[pl.ds(s0, _RPT)])
            if with_deg:
                pltpu.sync_copy(zdeg.at[pl.ds(s0, _RPT)],
                                deg_sh.at[pl.ds(s0, _RPT)])

        @pl.when(sid == _NS - 1)
        def _():
            s0 = (_NS - 1) * _RPT
            pltpu.sync_copy(zacc.at[pl.ds(s0, _RTL)],
                            acc_sh.at[pl.ds(s0, _RTL)])
            if with_deg:
                pltpu.sync_copy(zdeg.at[pl.ds(s0, _RTL)],
                                deg_sh.at[pl.ds(s0, _RTL)])

        plsc.subcore_barrier()

        rbase = wid * _RPW

        def lin_issue(r, b):
            pltpu.async_copy(ei.at[:, r], eiv[b], semL[b])
            pltpu.async_copy(basis.at[:, r], bv[b], semL[b])

        def lin_wait(b):
            pltpu.make_async_copy(ei.at[:, 0], eiv[b], semL[b]).wait()
            pltpu.make_async_copy(basis.at[:, 0], bv[b], semL[b]).wait()

        def g_issue(b):
            pltpu.async_copy(y.at[eiv[b].at[0]], rows[b], semG[b])

        def g_wait(b):
            pltpu.make_async_copy(y.at[eiv[b].at[0]], rows[b], semG[b]).wait()

        def s_issue(b):
            pltpu.async_copy(msg[b], acc_sh.at[dstc[b]], semS[b], add=True)
            if with_deg:
                pltpu.async_copy(ones_v, deg_sh.at[dstc[b]], semD[b],
                                 add=True)

        def s_wait(b):
            pltpu.make_async_copy(msg[b], acc_sh.at[dstc[b]], semS[b]).wait()
            if with_deg:
                pltpu.make_async_copy(ones_v, deg_sh.at[dstc[b]],
                                      semD[b]).wait()

        def compute(b):
            def group(g, c2):
                g16 = g * 16
                bb0 = bv[b][0, pl.ds(g16, 16)]
                bb1 = bv[b][1, pl.ds(g16, 16)]
                bb2 = bv[b][2, pl.ds(g16, 16)]
                bb3 = bv[b][3, pl.ds(g16, 16)]
                for j in range(16):
                    e = g16 + j
                    for h in range(F // 16):
                        m = rows[b][e, pl.ds(h * 16, 16)] * bb0[j]
                        m = m + rows[b][e, pl.ds(F + h * 16, 16)] * bb1[j]
                        m = m + rows[b][e, pl.ds(2 * F + h * 16, 16)] * bb2[j]
                        m = m + rows[b][e, pl.ds(3 * F + h * 16, 16)] * bb3[j]
                        msg[b][e, pl.ds(h * 16, 16)] = m
                return c2

            lax.fori_loop(0, _C // 16, group, 0)
            for k in range(_C // 16):
                dstc[b][pl.ds(k * 16, 16)] = eiv[b][1, pl.ds(k * 16, 16)]

        lin_issue(rbase, 0)
        lin_issue(rbase + 1, 1)
        lin_wait(0)
        g_issue(0)

        def outer(jo, carry):
            for b in range(_NBUF):
                j = jo * _NBUF + b
                bn = 1 - b

                @pl.when(j + 1 < _NP)
                def _():
                    lin_wait(bn)
                    g_issue(bn)

                g_wait(b)

                @pl.when(j >= 2)
                def _():
                    s_wait(b)

                compute(b)

                @pl.when(j + 2 < _NP)
                def _():
                    lin_issue(rbase + j + 2, b)

                s_issue(b)
            return carry

        lax.fori_loop(0, _NP // _NBUF, outer, 0)

        for b in range(_NBUF):
            s_wait(b)
        lin_issue(rbase + _RPW - 1, 0)
        lin_wait(0)
        g_issue(0)
        g_wait(0)
        compute(0)
        s_issue(0)
        s_wait(0)

        plsc.subcore_barrier()

        @pl.when(sid < _NS - 1)
        def _():
            s0 = sid * _RPT
            pltpu.sync_copy(acc_sh.at[pl.ds(s0, _RPT)],
                            acc_out.at[cid, pl.ds(s0, _RPT)])
            if with_deg:
                pltpu.sync_copy(deg_sh.at[pl.ds(s0, _RPT)],
                                deg_out.at[cid, pl.ds(s0, _RPT)])

        @pl.when(sid == _NS - 1)
        def _():
            s0 = (_NS - 1) * _RPT
            pltpu.sync_copy(acc_sh.at[pl.ds(s0, _RTL)],
                            acc_out.at[cid, pl.ds(s0, _RTL)])
            if with_deg:
                pltpu.sync_copy(deg_sh.at[pl.ds(s0, _RTL)],
                                deg_out.at[cid, pl.ds(s0, _RTL)])

    return pl.kernel(body, out_type=out_type, mesh=mesh,
                     scratch_types=scratch,
                     compiler_params=pltpu.CompilerParams(
                         use_tc_tiling_on_sc=False))


_edge1 = _make_edge_call(_HID, True)
_edge2 = _make_edge_call(_OUT_F, False)


def kernel(x, edge_index, edge_attr, batch, W1, gamma1, beta1,
           W2, gamma2, beta2, Wfc):
    f32 = jnp.float32
    basis = _basis_call(edge_attr.T)
    w1f = W1.transpose(1, 0, 2).reshape(_IN_F, _K * _HID)
    w2f = W2.transpose(1, 0, 2).reshape(_HID, _K * _OUT_F)
    y1 = _xw_call(x, w1f)

    ei3 = edge_index.reshape(2, _ER, _C)
    basis3 = basis.reshape(4, _ER, _C)
    z1 = jnp.zeros((_N, _HID), f32)
    zd = jnp.zeros((_N,), f32)
    acc1p, degp = _edge1(ei3, basis3, y1, z1, zd)
    degr = degp.reshape(_NC, _NBLK, 1, _NB)

    h1, st1 = _cs16(acc1p, degr)
    y2 = _m2_call(h1, st1, gamma1.reshape(1, _HID), beta1.reshape(1, _HID),
                  w2f)

    z2 = jnp.zeros((_N, _OUT_F), f32)
    acc2p = _edge2(ei3, basis3, y2, z2)

    h2, st2 = _cs32(acc2p, degr)
    out = _f2_call(h2, st2, gamma2.reshape(1, _OUT_F),
                   beta2.reshape(1, _OUT_F),
                   batch.reshape(_NBLK, 1, _NB), Wfc)
    return out


# speedup vs baseline: 20.2981x; 1.3027x over previous
import jax
import jax.numpy as jnp
from jax import lax
from jax.experimental import pallas as pl
from jax.experimental.pallas import tpu as pltpu
from jax.experimental.pallas import tpu_sc as plsc

_N = 50000
_E = 1600000
_G = 64
_IN_F = 3
_HID = 16
_OUT_F = 32
_NCLS = 10
_K = 4
_EPS = 1e-5

_NC = 2
_NS = 16
_NW = _NC * _NS
_EPW = _E // _NW
_C = 80
_NCH = _EPW // _C

_RPT = 3200
_RTL = _N - _RPT * (_NS - 1)

_NB = 2000
_NBLK = _N // _NB
_EBB = 3200
_EBLK = _E // _EBB


def _xw_body(x_ref, w_ref, y_ref):
    y_ref[...] = jnp.dot(x_ref[...], w_ref[...],
                         preferred_element_type=jnp.float32)


_xw_call = pl.pallas_call(
    _xw_body,
    grid=(_NBLK,),
    in_specs=[pl.BlockSpec((_NB, _IN_F), lambda i: (i, 0)),
              pl.BlockSpec((_IN_F, _K * _HID), lambda i: (0, 0))],
    out_specs=pl.BlockSpec((_NB, _K * _HID), lambda i: (i, 0)),
    out_shape=jax.ShapeDtypeStruct((_N, _K * _HID), jnp.float32),
)


def _bn_scale_shift(st_ref, g_ref, b_ref):
    mean = st_ref[0:1, :] * (1.0 / _N)
    var = st_ref[1:2, :] * (1.0 / _N) - mean * mean
    scale = g_ref[...] * lax.rsqrt(var + _EPS)
    shift = b_ref[...] - mean * scale
    return scale, shift


def _elu_mean(acc_ref, deg_ref):
    acc = acc_ref[0] + acc_ref[1]
    deg = deg_ref[0, 0, 0, :] + deg_ref[1, 0, 0, :]
    deg = jnp.maximum(deg, 1.0)
    h = acc / deg[:, None]
    return jnp.where(h > 0.0, h, jnp.exp(h) - 1.0)


def _mid_body(acc_ref, deg_ref, g_ref, b_ref, w_ref, y_ref, h_sc, st_sc):
    p = pl.program_id(0)
    i = pl.program_id(1)

    @pl.when(p == 0)
    def _():
        h = _elu_mean(acc_ref, deg_ref)
        h_sc[pl.ds(i * _NB, _NB), :] = h

        @pl.when(i == 0)
        def _():
            st_sc[...] = jnp.zeros((2, _HID), jnp.float32)

        st_sc[0:1, :] += jnp.sum(h, axis=0, keepdims=True)
        st_sc[1:2, :] += jnp.sum(h * h, axis=0, keepdims=True)

    @pl.when(p == 1)
    def _():
        scale, shift = _bn_scale_shift(st_sc, g_ref, b_ref)
        hb = h_sc[pl.ds(i * _NB, _NB), :] * scale + shift
        y_ref[...] = jnp.dot(hb, w_ref[...],
                             preferred_element_type=jnp.float32)


_mid_call = pl.pallas_call(
    _mid_body,
    grid=(2, _NBLK),
    in_specs=[pl.BlockSpec((2, _NB, _HID), lambda p, i: (0, i, 0)),
              pl.BlockSpec((2, 1, 1, _NB), lambda p, i: (0, i, 0, 0)),
              pl.BlockSpec((1, _HID), lambda p, i: (0, 0)),
              pl.BlockSpec((1, _HID), lambda p, i: (0, 0)),
              pl.BlockSpec((_HID, _K * _OUT_F), lambda p, i: (0, 0))],
    out_specs=pl.BlockSpec((_NB, _K * _OUT_F), lambda p, i: (i, 0)),
    out_shape=jax.ShapeDtypeStruct((_N, _K * _OUT_F), jnp.float32),
    scratch_shapes=[pltpu.VMEM((_N, _HID), jnp.float32),
                    pltpu.VMEM((2, _HID), jnp.float32)],
)


def _fin_body(acc_ref, deg_ref, g_ref, b_ref, batch_ref, wfc_ref, o_ref,
              h_sc, st_sc, ps_ref, cnt_ref):
    p = pl.program_id(0)
    i = pl.program_id(1)

    @pl.when(p == 0)
    def _():
        h = _elu_mean(acc_ref, deg_ref)
        h_sc[pl.ds(i * _NB, _NB), :] = h

        @pl.when(i == 0)
        def _():
            st_sc[...] = jnp.zeros((2, _OUT_F), jnp.float32)

        st_sc[0:1, :] += jnp.sum(h, axis=0, keepdims=True)
        st_sc[1:2, :] += jnp.sum(h * h, axis=0, keepdims=True)

    @pl.when(p == 1)
    def _():
        scale, shift = _bn_scale_shift(st_sc, g_ref, b_ref)
        hb = h_sc[pl.ds(i * _NB, _NB), :] * scale + shift
        bt = batch_ref[0, 0, :]
        oh = (bt[:, None] ==
              lax.broadcasted_iota(jnp.int32, (_NB, _G), 1)
              ).astype(jnp.float32)

        @pl.when(i == 0)
        def _():
            ps_ref[...] = jnp.zeros((_G, _OUT_F), jnp.float32)
            cnt_ref[...] = jnp.zeros((_G, _OUT_F), jnp.float32)

        dn = (((0,), (0,)), ((), ()))
        ps_ref[...] += lax.dot_general(oh, hb, dn,
                                       preferred_element_type=jnp.float32)
        cnt_ref[...] += lax.dot_general(
            oh, jnp.ones((_NB, _OUT_F), jnp.float32), dn,
            preferred_element_type=jnp.float32)

        @pl.when(i == _NBLK - 1)
        def _():
            pooled = ps_ref[...] / jnp.maximum(cnt_ref[...], 1.0)
            o_ref[...] = jnp.dot(pooled, wfc_ref[...],
                                 preferred_element_type=jnp.float32)


_fin_call = pl.pallas_call(
    _fin_body,
    grid=(2, _NBLK),
    in_specs=[pl.BlockSpec((2, _NB, _OUT_F), lambda p, i: (0, i, 0)),
              pl.BlockSpec((2, 1, 1, _NB), lambda p, i: (0, i, 0, 0)),
              pl.BlockSpec((1, _OUT_F), lambda p, i: (0, 0)),
              pl.BlockSpec((1, _OUT_F), lambda p, i: (0, 0)),
              pl.BlockSpec((1, 1, _NB), lambda p, i: (i, 0, 0)),
              pl.BlockSpec((_OUT_F, _NCLS), lambda p, i: (0, 0))],
    out_specs=pl.BlockSpec((_G, _NCLS), lambda p, i: (0, 0)),
    out_shape=jax.ShapeDtypeStruct((_G, _NCLS), jnp.float32),
    scratch_shapes=[pltpu.VMEM((_N, _OUT_F), jnp.float32),
                    pltpu.VMEM((2, _OUT_F), jnp.float32),
                    pltpu.VMEM((_G, _OUT_F), jnp.float32),
                    pltpu.VMEM((_G, _OUT_F), jnp.float32)],
)


_NBUF = 2
_ER = _E // _C
_RPW = _ER // _NW
_NP = (_RPW - 1) // _NBUF * _NBUF


def _make_edge_call(F, with_deg):
    KF = _K * F
    mesh = plsc.VectorSubcoreMesh(core_axis_name="c", subcore_axis_name="s",
                                  num_cores=_NC, num_subcores=_NS)
    if with_deg:
        out_type = (jax.ShapeDtypeStruct((_NC, _N, F), jnp.float32),
                    jax.ShapeDtypeStruct((_NC, _N), jnp.float32))
    else:
        out_type = jax.ShapeDtypeStruct((_NC, _N, F), jnp.float32)

    scratch = []
    scratch += [pltpu.VMEM((2, _C), jnp.int32)] * _NBUF
    scratch += [pltpu.VMEM((2, _C), jnp.float32)] * _NBUF
    scratch += [pltpu.VMEM((_C, KF), jnp.float32)] * _NBUF
    scratch += [pltpu.VMEM((_C, F), jnp.float32)] * _NBUF
    scratch += [pltpu.VMEM((_C,), jnp.int32)] * _NBUF
    scratch.append(pltpu.VMEM((_C,), jnp.float32))
    scratch.append(pltpu.VMEM((_C, F), jnp.float32))
    scratch.append(pltpu.VMEM_SHARED((_N, F), jnp.float32))
    if with_deg:
        scratch.append(pltpu.VMEM((_C,), jnp.float32))
        scratch.append(pltpu.VMEM_SHARED((_N,), jnp.float32))
    nsem = 3 * _NBUF + (_NBUF if with_deg else 0)
    scratch += [pltpu.SemaphoreType.DMA] * nsem

    def body(*refs):
        if with_deg:
            (ei, eat, y, acc_out, deg_out) = refs[:5]
            rest = refs[5:]
        else:
            (ei, eat, y, acc_out) = refs[:4]
            rest = refs[4:]
        eiv = rest[0:_NBUF]
        eav = rest[_NBUF:2 * _NBUF]
        rows = rest[2 * _NBUF:3 * _NBUF]
        msg = rest[3 * _NBUF:4 * _NBUF]
        dstc = rest[4 * _NBUF:5 * _NBUF]
        ones_v = rest[5 * _NBUF]
        zbuf = rest[5 * _NBUF + 1]
        acc_sh = rest[5 * _NBUF + 2]
        pos = 5 * _NBUF + 3
        if with_deg:
            zdbuf = rest[pos]
            deg_sh = rest[pos + 1]
            pos += 2
        semL = rest[pos:pos + _NBUF]
        semG = rest[pos + _NBUF:pos + 2 * _NBUF]
        semS = rest[pos + 2 * _NBUF:pos + 3 * _NBUF]
        if with_deg:
            semD = rest[pos + 3 * _NBUF:pos + 4 * _NBUF]

        cid = lax.axis_index("c")
        sid = lax.axis_index("s")
        wid = sid * _NC + cid

        for k in range(_C // 16):
            ones_v[pl.ds(k * 16, 16)] = jnp.ones((16,), jnp.float32)

        def zrow(r, c):
            for h in range(F // 16):
                zbuf[r, pl.ds(h * 16, 16)] = jnp.zeros((16,), jnp.float32)
            return c

        lax.fori_loop(0, _C, zrow, 0)
        if with_deg:
            for k in range(_C // 16):
                zdbuf[pl.ds(k * 16, 16)] = jnp.zeros((16,), jnp.float32)
        s0 = sid * _RPT
        nz = jnp.where(sid < _NS - 1, _RPT // _C, _RTL // _C)

        def zcopy(q, c):
            pltpu.sync_copy(zbuf, acc_sh.at[pl.ds(s0 + q * _C, _C)])
            if with_deg:
                pltpu.sync_copy(zdbuf, deg_sh.at[pl.ds(s0 + q * _C, _C)])
            return c

        lax.fori_loop(0, nz, zcopy, 0)

        plsc.subcore_barrier()

        rbase = wid * _RPW

        def lin_issue(r, b):
            pltpu.async_copy(ei.at[:, r], eiv[b], semL[b])
            pltpu.async_copy(eat.at[:, r], eav[b], semL[b])

        def lin_wait(b):
            pltpu.make_async_copy(ei.at[:, 0], eiv[b], semL[b]).wait()
            pltpu.make_async_copy(eat.at[:, 0], eav[b], semL[b]).wait()

        def g_issue(b):
            pltpu.async_copy(y.at[eiv[b].at[0]], rows[b], semG[b])

        def g_wait(b):
            pltpu.make_async_copy(y.at[eiv[b].at[0]], rows[b], semG[b]).wait()

        def s_issue(b):
            pltpu.async_copy(msg[b], acc_sh.at[dstc[b]], semS[b], add=True)
            if with_deg:
                pltpu.async_copy(ones_v, deg_sh.at[dstc[b]], semD[b],
                                 add=True)

        def s_wait(b):
            pltpu.make_async_copy(msg[b], acc_sh.at[dstc[b]], semS[b]).wait()
            if with_deg:
                pltpu.make_async_copy(ones_v, deg_sh.at[dstc[b]],
                                      semD[b]).wait()

        def compute(b):
            def group(g, c2):
                g16 = g * 16
                f0 = jnp.clip(eav[b][0, pl.ds(g16, 16)], 0.0, 1.0)
                f1 = jnp.clip(eav[b][1, pl.ds(g16, 16)], 0.0, 1.0)
                bb3 = f0 * f1
                bb1 = f0 - bb3
                bb2 = f1 - bb3
                bb0 = (1.0 - f0) - bb2
                for j in range(16):
                    e = g16 + j
                    for h in range(F // 16):
                        m = rows[b][e, pl.ds(h * 16, 16)] * bb0[j]
                        m = m + rows[b][e, pl.ds(F + h * 16, 16)] * bb1[j]
                        m = m + rows[b][e, pl.ds(2 * F + h * 16, 16)] * bb2[j]
                        m = m + rows[b][e, pl.ds(3 * F + h * 16, 16)] * bb3[j]
                        msg[b][e, pl.ds(h * 16, 16)] = m
                return c2

            lax.fori_loop(0, _C // 16, group, 0)
            for k in range(_C // 16):
                dstc[b][pl.ds(k * 16, 16)] = eiv[b][1, pl.ds(k * 16, 16)]

        lin_issue(rbase, 0)
        lin_issue(rbase + 1, 1)
        lin_wait(0)
        g_issue(0)

        def outer(jo, carry):
            for b in range(_NBUF):
                j = jo * _NBUF + b
                bn = 1 - b

                @pl.when(j + 1 < _NP)
                def _():
                    lin_wait(bn)
                    g_issue(bn)

                g_wait(b)

                @pl.when(j >= 2)
                def _():
                    s_wait(b)

                compute(b)

                @pl.when(j + 2 < _NP)
                def _():
                    lin_issue(rbase + j + 2, b)

                s_issue(b)
            return carry

        lax.fori_loop(0, _NP // _NBUF, outer, 0)

        for b in range(_NBUF):
            s_wait(b)
        lin_issue(rbase + _RPW - 1, 0)
        lin_wait(0)
        g_issue(0)
        g_wait(0)
        compute(0)
        s_issue(0)
        s_wait(0)

        plsc.subcore_barrier()

        @pl.when(sid < _NS - 1)
        def _():
            s0 = sid * _RPT
            pltpu.sync_copy(acc_sh.at[pl.ds(s0, _RPT)],
                            acc_out.at[cid, pl.ds(s0, _RPT)])
            if with_deg:
                pltpu.sync_copy(deg_sh.at[pl.ds(s0, _RPT)],
                                deg_out.at[cid, pl.ds(s0, _RPT)])

        @pl.when(sid == _NS - 1)
        def _():
            s0 = (_NS - 1) * _RPT
            pltpu.sync_copy(acc_sh.at[pl.ds(s0, _RTL)],
                            acc_out.at[cid, pl.ds(s0, _RTL)])
            if with_deg:
                pltpu.sync_copy(deg_sh.at[pl.ds(s0, _RTL)],
                                deg_out.at[cid, pl.ds(s0, _RTL)])

    return pl.kernel(body, out_type=out_type, mesh=mesh,
                     scratch_types=scratch,
                     compiler_params=pltpu.CompilerParams(
                         use_tc_tiling_on_sc=False))


_edge1 = _make_edge_call(_HID, True)
_edge2 = _make_edge_call(_OUT_F, False)


def kernel(x, edge_index, edge_attr, batch, W1, gamma1, beta1,
           W2, gamma2, beta2, Wfc):
    w1f = W1.transpose(1, 0, 2).reshape(_IN_F, _K * _HID)
    w2f = W2.transpose(1, 0, 2).reshape(_HID, _K * _OUT_F)
    y1 = _xw_call(x, w1f)

    ei3 = edge_index.reshape(2, _ER, _C)
    eat3 = edge_attr.T.reshape(2, _ER, _C)
    acc1p, degp = _edge1(ei3, eat3, y1)
    degr = degp.reshape(_NC, _NBLK, 1, _NB)

    y2 = _mid_call(acc1p, degr, gamma1.reshape(1, _HID),
                   beta1.reshape(1, _HID), w2f)

    acc2p = _edge2(ei3, eat3, y2)

    out = _fin_call(acc2p, degr, gamma2.reshape(1, _OUT_F),
                    beta2.reshape(1, _OUT_F),
                    batch.reshape(_NBLK, 1, _NB), Wfc)
    return out
